# baseline (speedup 1.0000x reference)
import functools

import jax
import jax.numpy as jnp
import numpy as np
from jax import lax
from jax.experimental import pallas as pl
from jax.experimental.pallas import tpu as pltpu
from jax.experimental.pallas import tpu_sc as plsc

BATCH = 64
SEQ = 2048
D = 128
LANES = 16
NUM_WORKERS = 32
POS_PER_W = SEQ // NUM_WORKERS
SCALE = float(np.sqrt(float(D)))


def _positional_encoding(length, depth):
    half = depth // 2
    positions = np.arange(length)[:, None].astype(np.float32)
    depths = (np.arange(half)[None, :] / float(half)).astype(np.float32)
    angle_rates = 1.0 / (10000.0 ** depths)
    angle_rads = positions * angle_rates
    return np.concatenate([np.sin(angle_rads), np.cos(angle_rads)], axis=-1)


_PE = _positional_encoding(SEQ, D)


def _sc_kernel(x_hbm, table_hbm, pe_hbm, out_hbm,
               idx_v, rows0, rows1, pe_v, sem0, sem1):
    nc = 2
    wid = lax.axis_index("s") * nc + lax.axis_index("c")
    pos_base = wid * POS_PER_W

    pltpu.sync_copy(pe_hbm.at[pl.ds(pos_base, POS_PER_W)], pe_v)

    def load_idx(b, _):
        pltpu.sync_copy(x_hbm.at[b, pl.ds(pos_base, POS_PER_W)], idx_v.at[b])
        return 0
    lax.fori_loop(0, BATCH, load_idx, 0)

    bufs = (rows0, rows1)
    sems = (sem0, sem1)

    def start_gather(b, p):
        pltpu.make_async_copy(
            table_hbm.at[idx_v.at[b]], bufs[p], sems[p]).start()

    def wait_gather(b, p):
        pltpu.make_async_copy(
            table_hbm.at[idx_v.at[b]], bufs[p], sems[p]).wait()

    start_gather(0, 0)
    start_gather(1, 1)

    def compute(buf):
        def row_body(r, _):
            for j in range(D // LANES):
                sl = pl.ds(j * LANES, LANES)
                buf[r, sl] = buf[r, sl] * SCALE + pe_v[r, sl]
            return 0
        lax.fori_loop(0, POS_PER_W, row_body, 0)

    def step(b0, _):
        for p in range(2):
            b = b0 + p
            wait_gather(b, p)
            compute(bufs[p])
            pltpu.sync_copy(bufs[p], out_hbm.at[b, pl.ds(pos_base, POS_PER_W)])

            @pl.when(b + 2 < BATCH)
            def _():
                start_gather(b + 2, p)
        return 0

    lax.fori_loop(0, BATCH // 2, lambda i, c: step(i * 2, c), 0)


def kernel(x, table):
    x = x.astype(jnp.int32)
    pe = jnp.asarray(_PE, dtype=jnp.float32)
    mesh = plsc.VectorSubcoreMesh(core_axis_name="c", subcore_axis_name="s")
    k = functools.partial(
        pl.kernel,
        mesh=mesh,
        out_type=jax.ShapeDtypeStruct((BATCH, SEQ, D), jnp.float32),
        scratch_types=[
            pltpu.VMEM((BATCH, POS_PER_W), jnp.int32),
            pltpu.VMEM((POS_PER_W, D), jnp.float32),
            pltpu.VMEM((POS_PER_W, D), jnp.float32),
            pltpu.VMEM((POS_PER_W, D), jnp.float32),
            pltpu.SemaphoreType.DMA,
            pltpu.SemaphoreType.DMA,
        ],
    )(_sc_kernel)
    return k(x, table, pe)


# speedup vs baseline: 1.6572x; 1.6572x over previous
import functools

import jax
import jax.numpy as jnp
import numpy as np
from jax import lax
from jax.experimental import pallas as pl
from jax.experimental.pallas import tpu as pltpu
from jax.experimental.pallas import tpu_sc as plsc

BATCH = 64
SEQ = 2048
D = 128
LANES = 16
NUM_WORKERS = 32
POS_PER_W = SEQ // NUM_WORKERS
SCALE = float(np.sqrt(float(D)))
NBUF = 4
NCHUNK = BATCH // 2


def _positional_encoding(length, depth):
    half = depth // 2
    positions = np.arange(length)[:, None].astype(np.float32)
    depths = (np.arange(half)[None, :] / float(half)).astype(np.float32)
    angle_rates = 1.0 / (10000.0 ** depths)
    angle_rads = positions * angle_rates
    return np.concatenate([np.sin(angle_rads), np.cos(angle_rads)], axis=-1)


_PE = _positional_encoding(SEQ, D)


def _sc_kernel(x_hbm, table_hbm, pe_hbm, out_hbm,
               idx_v, b0, b1, b2, b3, pe_v,
               g0, g1, g2, g3, s0, s1, s2, s3):
    nc = 2
    wid = lax.axis_index("s") * nc + lax.axis_index("c")
    pos_base = wid * POS_PER_W

    bufs = (b0, b1, b2, b3)
    gsems = (g0, g1, g2, g3)
    ssems = (s0, s1, s2, s3)

    def idx_copy(b):
        return pltpu.make_async_copy(
            x_hbm.at[b, pl.ds(pos_base, POS_PER_W)], idx_v.at[b], s0)

    def issue_idx(b, _):
        idx_copy(b).start()
        return 0

    def drain_idx(b, _):
        idx_copy(b).wait()
        return 0

    lax.fori_loop(0, BATCH, issue_idx, 0)
    pltpu.sync_copy(pe_hbm.at[pl.ds(pos_base, POS_PER_W)], pe_v)
    lax.fori_loop(0, BATCH, drain_idx, 0)

    def gather_copies(c, p):
        buf = bufs[p]
        return (
            pltpu.make_async_copy(
                table_hbm.at[idx_v.at[2 * c]],
                buf.at[pl.ds(0, POS_PER_W)], gsems[p]),
            pltpu.make_async_copy(
                table_hbm.at[idx_v.at[2 * c + 1]],
                buf.at[pl.ds(POS_PER_W, POS_PER_W)], gsems[p]),
        )

    def scatter_copies(c, p):
        buf = bufs[p]
        return (
            pltpu.make_async_copy(
                buf.at[pl.ds(0, POS_PER_W)],
                out_hbm.at[2 * c, pl.ds(pos_base, POS_PER_W)], ssems[p]),
            pltpu.make_async_copy(
                buf.at[pl.ds(POS_PER_W, POS_PER_W)],
                out_hbm.at[2 * c + 1, pl.ds(pos_base, POS_PER_W)], ssems[p]),
        )

    def start(copies):
        for cp in copies:
            cp.start()

    def wait(copies):
        for cp in copies:
            cp.wait()

    def compute(buf):
        def row_body(r, _):
            for j in range(D // LANES):
                sl = pl.ds(j * LANES, LANES)
                pv = pe_v[r, sl]
                buf[r, sl] = buf[r, sl] * SCALE + pv
                r2 = r + POS_PER_W
                buf[r2, sl] = buf[r2, sl] * SCALE + pv
            return 0
        lax.fori_loop(0, POS_PER_W, row_body, 0)

    start(gather_copies(0, 0))
    start(gather_copies(1, 1))

    def step(c0, _):
        for p in range(NBUF):
            c = c0 + p
            wait(gather_copies(c, p))
            compute(bufs[p])
            start(scatter_copies(c, p))

            @pl.when(c + 2 < NCHUNK)
            def _():
                pn = (p + 2) % NBUF

                @pl.when(c - 2 >= 0)
                def _():
                    wait(scatter_copies(c - 2, pn))
                start(gather_copies(c + 2, pn))
        return 0

    lax.fori_loop(0, NCHUNK // NBUF, lambda i, cr: step(i * NBUF, cr), 0)

    for c in range(NCHUNK - 4, NCHUNK):
        wait(scatter_copies(c, c % NBUF))


def kernel(x, table):
    x = x.astype(jnp.int32)
    pe = jnp.asarray(_PE, dtype=jnp.float32)
    mesh = plsc.VectorSubcoreMesh(core_axis_name="c", subcore_axis_name="s")
    k = functools.partial(
        pl.kernel,
        mesh=mesh,
        out_type=jax.ShapeDtypeStruct((BATCH, SEQ, D), jnp.float32),
        scratch_types=[
            pltpu.VMEM((BATCH, POS_PER_W), jnp.int32),
            pltpu.VMEM((2 * POS_PER_W, D), jnp.float32),
            pltpu.VMEM((2 * POS_PER_W, D), jnp.float32),
            pltpu.VMEM((2 * POS_PER_W, D), jnp.float32),
            pltpu.VMEM((2 * POS_PER_W, D), jnp.float32),
            pltpu.VMEM((POS_PER_W, D), jnp.float32),
            pltpu.SemaphoreType.DMA,
            pltpu.SemaphoreType.DMA,
            pltpu.SemaphoreType.DMA,
            pltpu.SemaphoreType.DMA,
            pltpu.SemaphoreType.DMA,
            pltpu.SemaphoreType.DMA,
            pltpu.SemaphoreType.DMA,
            pltpu.SemaphoreType.DMA,
        ],
    )(_sc_kernel)
    return k(x, table, pe)


# speedup vs baseline: 1.6743x; 1.0103x over previous
import functools

import jax
import jax.numpy as jnp
import numpy as np
from jax import lax
from jax.experimental import pallas as pl
from jax.experimental.pallas import tpu as pltpu
from jax.experimental.pallas import tpu_sc as plsc

BATCH = 64
SEQ = 2048
D = 128
LANES = 16
NUM_WORKERS = 32
POS_PER_W = SEQ // NUM_WORKERS
SCALE = float(np.sqrt(float(D)))
NBUF = 4
NCHUNK = BATCH // 2


def _positional_encoding(length, depth):
    half = depth // 2
    positions = np.arange(length)[:, None].astype(np.float32)
    depths = (np.arange(half)[None, :] / float(half)).astype(np.float32)
    angle_rates = 1.0 / (10000.0 ** depths)
    angle_rads = positions * angle_rates
    return np.concatenate([np.sin(angle_rads), np.cos(angle_rads)], axis=-1)


_PE = _positional_encoding(SEQ, D)


def _sc_kernel(x_hbm, table_hbm, pe_hbm, out_hbm,
               idx_v, b0, b1, b2, b3, pe_v,
               g0, g1, g2, g3, s0, s1, s2, s3):
    nc = 2
    wid = lax.axis_index("s") * nc + lax.axis_index("c")
    pos_base = wid * POS_PER_W

    bufs = (b0, b1, b2, b3)
    gsems = (g0, g1, g2, g3)
    ssems = (s0, s1, s2, s3)

    def idx_copy(b):
        return pltpu.make_async_copy(
            x_hbm.at[b, pl.ds(pos_base, POS_PER_W)], idx_v.at[b], s0)

    def issue_idx(b, _):
        idx_copy(b).start()
        return 0

    def drain_idx(b, _):
        idx_copy(b).wait()
        return 0

    lax.fori_loop(0, 4, issue_idx, 0)
    lax.fori_loop(0, 4, drain_idx, 0)

    def gather_copies(c, p):
        buf = bufs[p]
        return (
            pltpu.make_async_copy(
                table_hbm.at[idx_v.at[2 * c]],
                buf.at[pl.ds(0, POS_PER_W)], gsems[p]),
            pltpu.make_async_copy(
                table_hbm.at[idx_v.at[2 * c + 1]],
                buf.at[pl.ds(POS_PER_W, POS_PER_W)], gsems[p]),
        )

    def scatter_copies(c, p):
        buf = bufs[p]
        return (
            pltpu.make_async_copy(
                buf.at[pl.ds(0, POS_PER_W)],
                out_hbm.at[2 * c, pl.ds(pos_base, POS_PER_W)], ssems[p]),
            pltpu.make_async_copy(
                buf.at[pl.ds(POS_PER_W, POS_PER_W)],
                out_hbm.at[2 * c + 1, pl.ds(pos_base, POS_PER_W)], ssems[p]),
        )

    def start(copies):
        for cp in copies:
            cp.start()

    def wait(copies):
        for cp in copies:
            cp.wait()

    def compute(buf):
        def row_body(r, _):
            for j in range(D // LANES):
                sl = pl.ds(j * LANES, LANES)
                pv = pe_v[r, sl]
                buf[r, sl] = buf[r, sl] * SCALE + pv
                r2 = r + POS_PER_W
                buf[r2, sl] = buf[r2, sl] * SCALE + pv
            return 0
        lax.fori_loop(0, POS_PER_W, row_body, 0)

    start(gather_copies(0, 0))
    start(gather_copies(1, 1))
    lax.fori_loop(4, BATCH, issue_idx, 0)
    pltpu.sync_copy(pe_hbm.at[pl.ds(pos_base, POS_PER_W)], pe_v)
    lax.fori_loop(4, BATCH, drain_idx, 0)

    def step(c0, _):
        for p in range(NBUF):
            c = c0 + p
            wait(gather_copies(c, p))
            compute(bufs[p])
            start(scatter_copies(c, p))

            @pl.when(c + 2 < NCHUNK)
            def _():
                pn = (p + 2) % NBUF

                @pl.when(c - 2 >= 0)
                def _():
                    wait(scatter_copies(c - 2, pn))
                start(gather_copies(c + 2, pn))
        return 0

    lax.fori_loop(0, NCHUNK // NBUF, lambda i, cr: step(i * NBUF, cr), 0)

    for c in range(NCHUNK - 4, NCHUNK):
        wait(scatter_copies(c, c % NBUF))


def kernel(x, table):
    x = x.astype(jnp.int32)
    pe = jnp.asarray(_PE, dtype=jnp.float32)
    mesh = plsc.VectorSubcoreMesh(core_axis_name="c", subcore_axis_name="s")
    k = functools.partial(
        pl.kernel,
        mesh=mesh,
        out_type=jax.ShapeDtypeStruct((BATCH, SEQ, D), jnp.float32),
        scratch_types=[
            pltpu.VMEM((BATCH, POS_PER_W), jnp.int32),
            pltpu.VMEM((2 * POS_PER_W, D), jnp.float32),
            pltpu.VMEM((2 * POS_PER_W, D), jnp.float32),
            pltpu.VMEM((2 * POS_PER_W, D), jnp.float32),
            pltpu.VMEM((2 * POS_PER_W, D), jnp.float32),
            pltpu.VMEM((POS_PER_W, D), jnp.float32),
            pltpu.SemaphoreType.DMA,
            pltpu.SemaphoreType.DMA,
            pltpu.SemaphoreType.DMA,
            pltpu.SemaphoreType.DMA,
            pltpu.SemaphoreType.DMA,
            pltpu.SemaphoreType.DMA,
            pltpu.SemaphoreType.DMA,
            pltpu.SemaphoreType.DMA,
        ],
    )(_sc_kernel)
    return k(x, table, pe)
